# baseline (speedup 1.0000x reference)
import functools

import jax
import jax.numpy as jnp
from jax import lax
from jax.experimental import pallas as pl
from jax.experimental.pallas import tpu as pltpu
from jax.experimental.pallas import tpu_sc as plsc

NC = 2
NS = 16
CH = 128
NQ = 4
KH = 2
KS = 2
HB = 4


def _sc_aggregate(n, n_acc, dq, ts):
  rt = n_acc // NS
  nt = n // NS
  sb = ts // HB
  kg = KH + KS

  mesh = plsc.VectorSubcoreMesh(core_axis_name="c", subcore_axis_name="s")

  @functools.partial(
      pl.kernel,
      out_type=[jax.ShapeDtypeStruct((n_acc, dq), jnp.float32)
                for _ in range(NQ)],
      mesh=mesh,
      scratch_types=[
          pltpu.VMEM((sb, CH), jnp.int32),
          pltpu.VMEM((sb, CH), jnp.int32),
          pltpu.VMEM((kg, CH, dq), jnp.float32),
          pltpu.VMEM_SHARED((n, dq), jnp.float32),
          pltpu.VMEM_SHARED((n_acc, dq), jnp.float32),
          pltpu.SemaphoreType.DMA,
          pltpu.SemaphoreType.DMA,
          pltpu.SemaphoreType.DMA,
      ],
      compiler_params=pltpu.CompilerParams(use_tc_tiling_on_sc=False),
  )
  def agg(colh, rowh, fflat, zrows,
          o0, o1, o2, o3, colv, rowv, gbuf, table, acc, semh, sems, semw):
    cid = lax.axis_index("c")
    sid = lax.axis_index("s")
    outs = (o0, o1, o2, o3)

    for p in range(2):
      q = 2 * cid + p

      pltpu.sync_copy(fflat.at[pl.ds(q * n + sid * nt, nt)],
                      table.at[pl.ds(sid * nt, nt)])
      zr = rt // 8
      for z in range(8):
        pltpu.sync_copy(zrows, acc.at[pl.ds(sid * rt + z * zr, zr)])
      plsc.subcore_barrier()

      for hb in range(HB):
        pltpu.sync_copy(colh.at[(q * NS + sid) * HB + hb], colv)
        pltpu.sync_copy(rowh.at[sid * HB + hb], rowv)

        def body(g, carry):
          j0 = kg * g
          hd = [pltpu.async_copy(fflat.at[colv.at[j0 + k]], gbuf.at[k],
                                 semh)
                for k in range(KH)]
          sd = [pltpu.async_copy(table.at[colv.at[j0 + KH + k]],
                                 gbuf.at[KH + k], sems)
                for k in range(KS)]
          for dsc in sd:
            dsc.wait()
          sc1 = [pltpu.async_copy(gbuf.at[KH + k],
                                  acc.at[rowv.at[j0 + KH + k]], semw,
                                  add=True)
                 for k in range(KS)]
          for dsc in hd:
            dsc.wait()
          for dsc in sc1:
            dsc.wait()
          sc2 = [pltpu.async_copy(gbuf.at[k], acc.at[rowv.at[j0 + k]], semw,
                                  add=True)
                 for k in range(KH)]
          for dsc in sc2:
            dsc.wait()
          return carry

        lax.fori_loop(0, sb // kg, body, 0)

      plsc.subcore_barrier()

      @pl.when(cid == 0)
      def _():
        pltpu.sync_copy(acc.at[pl.ds(sid * rt, rt)],
                        outs[p].at[pl.ds(sid * rt, rt)])

      @pl.when(cid == 1)
      def _():
        pltpu.sync_copy(acc.at[pl.ds(sid * rt, rt)],
                        outs[2 + p].at[pl.ds(sid * rt, rt)])

  return agg


def _tc_linear_body(a0, a1, a2, a3, w0, w1, w2, w3, bb, out):
  acc = jnp.dot(a0[...], w0[...], preferred_element_type=jnp.float32)
  acc += jnp.dot(a1[...], w1[...], preferred_element_type=jnp.float32)
  acc += jnp.dot(a2[...], w2[...], preferred_element_type=jnp.float32)
  acc += jnp.dot(a3[...], w3[...], preferred_element_type=jnp.float32)
  out[...] = acc + bb[...]


def kernel(edge_index, features, W, b):
  n, d = features.shape
  d_out = W.shape[0]
  e = edge_index.shape[1]
  dq = d // NQ
  kg = KH + KS

  ts = -(-e // (NS * CH * kg * HB)) * kg * HB
  e_pad = NS * ts * CH
  n_acc = -(-n // (NS * 8)) * (NS * 8)

  row = edge_index[0].astype(jnp.int32)
  col = edge_index[1].astype(jnp.int32)
  row = jnp.pad(row, (0, e_pad - e), constant_values=n)
  col = jnp.pad(col, (0, e_pad - e), constant_values=0)
  row3 = row.reshape(NS * HB, ts // HB, CH)
  col3 = col.reshape(NS * HB, ts // HB, CH)

  fflat = jnp.concatenate(
      [features[:, q * dq:(q + 1) * dq] for q in range(NQ)], axis=0)
  hbm_pos = (jnp.arange(ts // HB) % kg) < KH
  colh = jnp.concatenate(
      [jnp.where(hbm_pos[None, :, None], col3 + q * n, col3)
       for q in range(NQ)], axis=0)
  zrows = jnp.zeros((n_acc // NS // 8, dq), jnp.float32)

  aggs = _sc_aggregate(n, n_acc, dq, ts)(colh, row3, fflat, zrows)

  blk = 1000
  grid = n // blk
  wq = [W[:, q * dq:(q + 1) * dq].T for q in range(NQ)]
  bb = b.reshape(1, d_out)

  out = pl.pallas_call(
      _tc_linear_body,
      grid=(grid,),
      in_specs=(
          [pl.BlockSpec((blk, dq), lambda i: (i, 0)) for _ in range(NQ)]
          + [pl.BlockSpec((dq, d_out), lambda i: (0, 0)) for _ in range(NQ)]
          + [pl.BlockSpec((1, d_out), lambda i: (0, 0))]
      ),
      out_specs=pl.BlockSpec((blk, d_out), lambda i: (i, 0)),
      out_shape=jax.ShapeDtypeStruct((n, d_out), jnp.float32),
  )(*aggs, *wq, bb)

  return out


# speedup vs baseline: 1.8197x; 1.8197x over previous
import functools

import jax
import jax.numpy as jnp
from jax import lax
from jax.experimental import pallas as pl
from jax.experimental.pallas import tpu as pltpu
from jax.experimental.pallas import tpu_sc as plsc

NC = 2
NS = 16
CH = 128
NH = 2
K = 5
HB = 4


def _sc_aggregate(n, n_acc, dh, ts):
  rt = n_acc // NS
  nt = n // NS
  sb = ts // HB

  mesh = plsc.VectorSubcoreMesh(core_axis_name="c", subcore_axis_name="s")

  @functools.partial(
      pl.kernel,
      out_type=[jax.ShapeDtypeStruct((n_acc, dh), jnp.bfloat16)
                for _ in range(NH)],
      mesh=mesh,
      scratch_types=[
          pltpu.VMEM((sb, CH), jnp.int32),
          pltpu.VMEM((sb, CH), jnp.int32),
          pltpu.VMEM((K, CH, dh), jnp.bfloat16),
          pltpu.VMEM_SHARED((n, dh), jnp.bfloat16),
          pltpu.VMEM_SHARED((n_acc, dh), jnp.bfloat16),
          pltpu.SemaphoreType.DMA,
          pltpu.SemaphoreType.DMA,
      ],
      compiler_params=pltpu.CompilerParams(use_tc_tiling_on_sc=False),
  )
  def agg(colh, rowh, fflat, zrows,
          o0, o1, colv, rowv, gbuf, table, acc, sem0, sem1):
    cid = lax.axis_index("c")
    sid = lax.axis_index("s")
    outs = (o0, o1)

    pltpu.sync_copy(fflat.at[pl.ds(cid * n + sid * nt, nt)],
                    table.at[pl.ds(sid * nt, nt)])
    zr = rt // 8
    for z in range(8):
      pltpu.sync_copy(zrows, acc.at[pl.ds(sid * rt + z * zr, zr)])
    plsc.subcore_barrier()

    for hb in range(HB):
      pltpu.sync_copy(colh.at[sid * HB + hb], colv)
      pltpu.sync_copy(rowh.at[sid * HB + hb], rowv)

      def body(g, carry):
        j0 = K * g
        gds = [pltpu.async_copy(table.at[colv.at[j0 + k]], gbuf.at[k],
                                sem0)
               for k in range(K)]
        for dsc in gds:
          dsc.wait()
        sds = [pltpu.async_copy(gbuf.at[k], acc.at[rowv.at[j0 + k]], sem1,
                                add=True)
               for k in range(K)]
        for dsc in sds:
          dsc.wait()
        return carry

      lax.fori_loop(0, sb // K, body, 0)

    plsc.subcore_barrier()

    @pl.when(cid == 0)
    def _():
      pltpu.sync_copy(acc.at[pl.ds(sid * rt, rt)],
                      outs[0].at[pl.ds(sid * rt, rt)])

    @pl.when(cid == 1)
    def _():
      pltpu.sync_copy(acc.at[pl.ds(sid * rt, rt)],
                      outs[1].at[pl.ds(sid * rt, rt)])

  return agg


def _tc_linear_body(a0, a1, w0, w1, bb, out):
  acc = jnp.dot(a0[...].astype(jnp.float32), w0[...],
                preferred_element_type=jnp.float32)
  acc += jnp.dot(a1[...].astype(jnp.float32), w1[...],
                 preferred_element_type=jnp.float32)
  out[...] = acc + bb[...]


def kernel(edge_index, features, W, b):
  n, d = features.shape
  d_out = W.shape[0]
  e = edge_index.shape[1]
  dh = d // NH

  ts = -(-e // (NS * CH * K * HB)) * K * HB
  e_pad = NS * ts * CH
  n_acc = -(-n // (NS * 8)) * (NS * 8)

  row = edge_index[0].astype(jnp.int32)
  col = edge_index[1].astype(jnp.int32)
  row = jnp.pad(row, (0, e_pad - e), constant_values=n)
  col = jnp.pad(col, (0, e_pad - e), constant_values=0)
  row3 = row.reshape(NS * HB, ts // HB, CH)
  col3 = col.reshape(NS * HB, ts // HB, CH)

  fbf = features.astype(jnp.bfloat16)
  fflat = jnp.concatenate([fbf[:, :dh], fbf[:, dh:]], axis=0)
  zrows = jnp.zeros((n_acc // NS // 8, dh), jnp.bfloat16)

  agg0, agg1 = _sc_aggregate(n, n_acc, dh, ts)(col3, row3, fflat, zrows)

  blk = 1000
  grid = n // blk
  w0 = W[:, :dh].T
  w1 = W[:, dh:].T
  bb = b.reshape(1, d_out)

  out = pl.pallas_call(
      _tc_linear_body,
      grid=(grid,),
      in_specs=[
          pl.BlockSpec((blk, dh), lambda i: (i, 0)),
          pl.BlockSpec((blk, dh), lambda i: (i, 0)),
          pl.BlockSpec((dh, d_out), lambda i: (0, 0)),
          pl.BlockSpec((dh, d_out), lambda i: (0, 0)),
          pl.BlockSpec((1, d_out), lambda i: (0, 0)),
      ],
      out_specs=pl.BlockSpec((blk, d_out), lambda i: (i, 0)),
      out_shape=jax.ShapeDtypeStruct((n, d_out), jnp.float32),
  )(agg0, agg1, w0, w1, bb)

  return out
